# baseline (speedup 1.0000x reference)
import jax
import jax.numpy as jnp
from jax.experimental import pallas as pl

BATCH_BLOCK = 8


def _add_kernel(patches_ref, table_ref, out_ref):
    out_ref[...] = patches_ref[...] + table_ref[...]


def kernel(projected_patches, pos_embed_table):
    batch, num_patch, proj_dim = projected_patches.shape
    p_half = num_patch // 2
    grid = (batch // BATCH_BLOCK, 2)
    return pl.pallas_call(
        _add_kernel,
        grid=grid,
        in_specs=[
            pl.BlockSpec((BATCH_BLOCK, p_half, proj_dim), lambda i, j: (i, j, 0)),
            pl.BlockSpec((p_half, proj_dim), lambda i, j: (j, 0)),
        ],
        out_specs=pl.BlockSpec((BATCH_BLOCK, p_half, proj_dim), lambda i, j: (i, j, 0)),
        out_shape=jax.ShapeDtypeStruct(projected_patches.shape, projected_patches.dtype),
    )(projected_patches, pos_embed_table)


# speedup vs baseline: 1.0698x; 1.0698x over previous
import jax
import jax.numpy as jnp
from jax.experimental import pallas as pl
from jax.experimental.pallas import tpu as pltpu

BATCH = 128
NUM_PATCH = 576
PROJ_DIM = 768
BATCH_BLOCK = 4
NUM_CHUNKS = BATCH // BATCH_BLOCK
NBUF = 3

_BLOCK_T = pltpu.VMEM((BATCH_BLOCK, NUM_PATCH, PROJ_DIM), jnp.float32)


def _add_kernel(patches_hbm, table_hbm, out_hbm, tbl, ins, outs, tsem,
                gsem0, gsem1, gsem2, ssem0, ssem1, ssem2):
    gsems = (gsem0, gsem1, gsem2)
    ssems = (ssem0, ssem1, ssem2)

    def in_cp(c, k):
        return pltpu.make_async_copy(
            patches_hbm.at[pl.ds(c * BATCH_BLOCK, BATCH_BLOCK)],
            ins.at[k], gsems[k])

    def out_cp(c, k):
        return pltpu.make_async_copy(
            outs.at[k],
            out_hbm.at[pl.ds(c * BATCH_BLOCK, BATCH_BLOCK)], ssems[k])

    pltpu.make_async_copy(table_hbm, tbl, tsem).start()
    for k in range(NBUF):
        in_cp(k, k).start()
    pltpu.make_async_copy(table_hbm, tbl, tsem).wait()

    for c in range(NUM_CHUNKS):
        k = c % NBUF
        in_cp(c, k).wait()
        if c >= NBUF:
            out_cp(c - NBUF, k).wait()
        outs[k] = ins[k] + tbl[None]
        out_cp(c, k).start()
        if c + NBUF < NUM_CHUNKS:
            in_cp(c + NBUF, k).start()
    for c in range(NUM_CHUNKS - NBUF, NUM_CHUNKS):
        out_cp(c, c % NBUF).wait()


def kernel(projected_patches, pos_embed_table):
    return pl.pallas_call(
        _add_kernel,
        in_specs=[
            pl.BlockSpec(memory_space=pl.ANY),
            pl.BlockSpec(memory_space=pl.ANY),
        ],
        out_specs=pl.BlockSpec(memory_space=pl.ANY),
        out_shape=jax.ShapeDtypeStruct(projected_patches.shape, projected_patches.dtype),
        scratch_shapes=[
            pltpu.VMEM((NUM_PATCH, PROJ_DIM), jnp.float32),
            pltpu.VMEM((NBUF, BATCH_BLOCK, NUM_PATCH, PROJ_DIM), jnp.float32),
            pltpu.VMEM((NBUF, BATCH_BLOCK, NUM_PATCH, PROJ_DIM), jnp.float32),
            pltpu.SemaphoreType.DMA,
            pltpu.SemaphoreType.DMA, pltpu.SemaphoreType.DMA,
            pltpu.SemaphoreType.DMA,
            pltpu.SemaphoreType.DMA, pltpu.SemaphoreType.DMA,
            pltpu.SemaphoreType.DMA,
        ],
    )(projected_patches, pos_embed_table)
